# baseline (speedup 1.0000x reference)
import functools

import jax
import jax.numpy as jnp
from jax import lax
from jax.experimental import pallas as pl
from jax.experimental.pallas import tpu as pltpu
from jax.experimental.pallas import tpu_sc as plsc

B = 16384
D = 64
NC = 2
NS = 16
NW = NC * NS
BPW = B // NW
CHUNK = 128
NCHUNK = BPW // CHUNK

_SC_MESH = plsc.VectorSubcoreMesh(
    core_axis_name="c", subcore_axis_name="s", num_cores=NC, num_subcores=NS
)


@functools.partial(
    pl.kernel,
    out_type=(
        jax.ShapeDtypeStruct((B, D), jnp.float32),
        jax.ShapeDtypeStruct((B, D), jnp.float32),
    ),
    mesh=_SC_MESH,
    scratch_types=[
        pltpu.VMEM((NCHUNK, CHUNK), jnp.int32),
        pltpu.VMEM((NCHUNK, CHUNK), jnp.int32),
        pltpu.VMEM((NCHUNK, CHUNK, D), jnp.float32),
        pltpu.VMEM((NCHUNK, CHUNK, D), jnp.float32),
        pltpu.SemaphoreType.DMA,
        pltpu.SemaphoreType.DMA,
    ],
    compiler_params=pltpu.CompilerParams(use_tc_tiling_on_sc=False),
)
def _sc_gather(user3, movie3, uemb, memb, u_out, m_out,
               uidx_v, midx_v, urows, mrows, usem, msem):
    wid = lax.axis_index("s") * NC + lax.axis_index("c")
    base = wid * BPW
    pltpu.sync_copy(user3.at[wid], uidx_v)
    pltpu.sync_copy(movie3.at[wid], midx_v)
    ucopies = []
    mcopies = []
    for j in range(NCHUNK):
        ucopies.append(pltpu.async_copy(uemb.at[uidx_v.at[j]], urows.at[j], usem))
        mcopies.append(pltpu.async_copy(memb.at[midx_v.at[j]], mrows.at[j], msem))
    for j in range(NCHUNK):
        ucopies[j].wait()
        pltpu.sync_copy(urows.at[j], u_out.at[pl.ds(base + j * CHUNK, CHUNK)])
        mcopies[j].wait()
        pltpu.sync_copy(mrows.at[j], m_out.at[pl.ds(base + j * CHUNK, CHUNK)])


BLK = 2048


def _mlp_body(u_ref, m_ref, w1_ref, b1_ref, w2_ref, b2_ref, w3_ref, b3_ref,
              wo_ref, bo_ref, out_ref):
    dn = (((1,), (1,)), ((), ()))
    u = u_ref[...]
    m = m_ref[...]
    w1 = w1_ref[...]
    h = lax.dot_general(u, w1[:, :D], dn, preferred_element_type=jnp.float32)
    h = h + lax.dot_general(m, w1[:, D:], dn, preferred_element_type=jnp.float32)
    h = jnp.maximum(h + b1_ref[...], 0.0)
    h = lax.dot_general(h, w2_ref[...], dn, preferred_element_type=jnp.float32)
    h = jnp.maximum(h + b2_ref[...], 0.0)
    h = lax.dot_general(h, w3_ref[...], dn, preferred_element_type=jnp.float32)
    h = jnp.maximum(h + b3_ref[...], 0.0)
    out_ref[...] = jnp.sum(h * wo_ref[0, :], axis=1) + bo_ref[...]


def _full(shape):
    return pl.BlockSpec(shape, lambda i: tuple(0 for _ in shape))


_mlp = pl.pallas_call(
    _mlp_body,
    grid=(B // BLK,),
    in_specs=[
        pl.BlockSpec((BLK, D), lambda i: (i, 0)),
        pl.BlockSpec((BLK, D), lambda i: (i, 0)),
        _full((256, 2 * D)),
        _full((256,)),
        _full((128, 256)),
        _full((128,)),
        _full((64, 128)),
        _full((64,)),
        _full((1, 64)),
        _full((1,)),
    ],
    out_specs=pl.BlockSpec((BLK,), lambda i: (i,)),
    out_shape=jax.ShapeDtypeStruct((B,), jnp.float32),
)


def kernel(user, movie, user_emb, movie_emb, W1, b1, W2, b2, W3, b3, Wo, bo):
    user3 = user.astype(jnp.int32).reshape(NW, NCHUNK, CHUNK)
    movie3 = movie.astype(jnp.int32).reshape(NW, NCHUNK, CHUNK)
    u_rows, m_rows = _sc_gather(user3, movie3, user_emb, movie_emb)
    return _mlp(u_rows, m_rows, W1, b1, W2, b2, W3, b3, Wo, bo)
